# baseline (speedup 1.0000x reference)
import jax
import jax.numpy as jnp
from jax import lax
from jax.experimental import pallas as pl
from jax.experimental.pallas import tpu as pltpu
from jax.experimental.pallas import tpu_sc as plsc

B = 4096
L = 200
V = 100000
K = 300
DT = 304
DW = 256
NS = 2
CA = 104
CB = L - CA
NW = 32
SPW = B // NW
DJ = DW // 16
OG = 32


def _rne_hi16(u):
  return (u + 0x7FFF + ((u >> 16) & 1)) & jnp.uint32(0xFFFF0000)


def _transform_body(e_ref, w_ref, b_ref, t0_ref, t1_ref, et_ref):
  et_ref[...] = e_ref[...].T
  t = (
      jnp.dot(
          et_ref[...].astype(jnp.bfloat16),
          w_ref[...].astype(jnp.bfloat16),
          preferred_element_type=jnp.float32,
      )
      + b_ref[...]
  )
  u = lax.bitcast_convert_type(t, jnp.uint32)
  a0 = _rne_hi16(u[:, 0:128]) | (_rne_hi16(u[:, 128:256]) >> 16)
  tail = _rne_hi16(u[:, 256:304])
  a1 = jnp.concatenate(
      [tail, jnp.zeros((tail.shape[0], 80), jnp.uint32)], axis=1
  )
  t0_ref[...] = lax.bitcast_convert_type(a0, jnp.int32)
  t1_ref[...] = lax.bitcast_convert_type(a1, jnp.int32)


def _transform_table(embT, w1t_p, b1_p):
  mblk = 8192
  return pl.pallas_call(
      _transform_body,
      grid=(pl.cdiv(V, mblk),),
      in_specs=[
          pl.BlockSpec((K, mblk), lambda i: (0, i)),
          pl.BlockSpec((K, DT), lambda i: (0, 0)),
          pl.BlockSpec((1, DT), lambda i: (0, 0)),
      ],
      out_specs=[
          pl.BlockSpec((mblk, 128), lambda i: (i, 0)) for _ in range(NS)
      ],
      out_shape=[jax.ShapeDtypeStruct((V, 128), jnp.int32) for _ in range(NS)],
      scratch_shapes=[pltpu.VMEM((mblk, K), jnp.float32)],
  )(embT, w1t_p, b1_p)


def _pool_body(t0, t1, x_hbm, out_hbm,
               idx_v, r0, r1, out_v, sem0, sem1):
  wid = lax.axis_index("s") * 2 + lax.axis_index("c")
  ibase = pl.multiple_of(wid * (SPW * L), 8)
  obase = pl.multiple_of(wid * (SPW * DW), 8)
  tabs = (t0, t1)
  rows = (r0, r1)

  pltpu.sync_copy(x_hbm.at[pl.ds(ibase, SPW * L)], idx_v)

  sems = (sem0, sem1)
  cdesc = ((0, CA), (SPW * CA, CB))

  def issue(s, c):
    coff, cn = cdesc[c]
    off = pl.multiple_of(coff + s * cn, 8)
    for k in range(NS):
      pltpu.async_copy(
          tabs[k].at[idx_v.at[pl.ds(off, cn)]],
          rows[k].at[c, pl.ds(0, cn)],
          sems[c],
      )

  def wait(c):
    cn = cdesc[c][1]
    for k in range(NS):
      pltpu.make_async_copy(
          tabs[k].at[idx_v.at[pl.ds(0, cn)]],
          rows[k].at[c, pl.ds(0, cn)],
          sems[c],
      ).wait()

  issue(0, 0)
  issue(0, 1)

  RUN = 4
  zero32 = jnp.zeros((32,), jnp.bfloat16)

  def body(s, carry):
    accs = tuple(zero32 for _ in range(DJ))
    for c in range(2):
      wait(c)
      cn = cdesc[c][1]

      def rbody(i, a, c=c):
        a = list(a)
        for dr in range(RUN):
          r = i * RUN + dr
          for j in range(DJ):
            w = rows[j // 8][c, r, pl.ds(16 * (j % 8), 16)]
            a[j] = jnp.maximum(a[j], plsc.bitcast(w, jnp.bfloat16))
        return tuple(a)

      accs = lax.fori_loop(0, cn // RUN, rbody, accs)

      @pl.when(s + 1 < SPW)
      def _():
        issue(s + 1, c)

    off = (s % OG) * DW
    for j in range(DJ):
      out_v[pl.ds(off + 16 * j, 16)] = plsc.bitcast(accs[j], jnp.int32)

    @pl.when(s % OG == OG - 1)
    def _():
      goff = pl.multiple_of(obase + (s - (OG - 1)) * DW, 8)
      pltpu.sync_copy(out_v, out_hbm.at[pl.ds(goff, OG * DW)])

    return carry

  lax.fori_loop(0, SPW, body, 0)


def _pool(tables, x_flat):
  mesh = plsc.VectorSubcoreMesh(
      core_axis_name="c", subcore_axis_name="s", num_cores=2, num_subcores=16
  )
  k = pl.kernel(
      _pool_body,
      out_type=jax.ShapeDtypeStruct((B * DW,), jnp.int32),
      mesh=mesh,
      scratch_types=[
          pltpu.VMEM((SPW * L,), jnp.int32),
          pltpu.VMEM((2, CA, 128), jnp.int32),
          pltpu.VMEM((2, CA, 128), jnp.int32),
          pltpu.VMEM((OG * DW,), jnp.int32),
          pltpu.SemaphoreType.DMA,
          pltpu.SemaphoreType.DMA,
      ],
      compiler_params=pltpu.CompilerParams(
          use_tc_tiling_on_sc=False, needs_layout_passes=False
      ),
  )
  return k(tables[0], tables[1], x_flat)


def _fc2_body(h_ref, wa_ref, wb_ref, b_ref, o_ref):
  u = lax.bitcast_convert_type(h_ref[...], jnp.uint32)
  a = lax.bitcast_convert_type(u & jnp.uint32(0xFFFF0000), jnp.float32)
  bb = lax.bitcast_convert_type(u << 16, jnp.float32)
  o_ref[...] = (
      jnp.dot(a, wa_ref[...], preferred_element_type=jnp.float32)
      + jnp.dot(bb, wb_ref[...], preferred_element_type=jnp.float32)
      + b_ref[...]
  )


def _fc2(pool, wa, wb, b2_p):
  mblk = 1024
  return pl.pallas_call(
      _fc2_body,
      grid=(B // mblk,),
      in_specs=[
          pl.BlockSpec((mblk, DW), lambda i: (i, 0)),
          pl.BlockSpec((DW, 128), lambda i: (0, 0)),
          pl.BlockSpec((DW, 128), lambda i: (0, 0)),
          pl.BlockSpec((1, 128), lambda i: (0, 0)),
      ],
      out_specs=pl.BlockSpec((mblk, 128), lambda i: (i, 0)),
      out_shape=jax.ShapeDtypeStruct((B, 128), jnp.float32),
  )(pool, wa, wb, b2_p)


@jax.jit
def kernel(x, emb, W1, b1, W2, b2):
  w1t_p = jnp.pad(W1.T, ((0, 0), (0, DT - 300)))
  b1_p = jnp.pad(b1, (0, DT - 300)).reshape(1, DT)

  tables = _transform_table(emb.T, w1t_p, b1_p)

  x_i = x.astype(jnp.int32).reshape(NW, SPW, L)
  x_flat = jnp.concatenate(
      [x_i[:, :, :CA].reshape(NW, -1), x_i[:, :, CA:].reshape(NW, -1)], axis=1
  ).reshape(-1)

  pool = _pool(tables, x_flat).reshape(B, DW)

  w2p = jnp.pad(W2, ((0, 0), (0, 304 - 300)))
  wa = jnp.pad(
      jnp.concatenate([w2p[:, 0:128], w2p[:, 256:304]], axis=1).T,
      ((0, DW - 176), (0, 126)),
  )
  wb = jnp.pad(w2p[:, 128:256].T, ((0, DW - 128), (0, 126)))
  b2_p = jnp.pad(b2, (0, 126)).reshape(1, 128)
  out = _fc2(pool, wa, wb, b2_p)
  return out[:, :2]
